# Initial kernel scaffold; baseline (speedup 1.0000x reference)
import jax
import jax.numpy as jnp
from jax.experimental import pallas as pl


def kernel(x_user, x_tweet, params, edge_index_ut, edge_index_tu):
    raise NotImplementedError("write your pallas kernel here")


# speedup vs baseline: 10.3535x; 10.3535x over previous
import functools

import jax
import jax.numpy as jnp
from jax.experimental import pallas as pl


_N = 50000
_H = 2
_C = 32


def _mm_body(x_ref, w_ref, b_ref, o_ref):
    o_ref[...] = (
        jnp.dot(x_ref[...], w_ref[...], preferred_element_type=jnp.float32)
        + b_ref[...]
    )


def _mm(x, w, b, block_rows=5000):
    m, k = x.shape
    n = w.shape[1]
    grid = m // block_rows
    return pl.pallas_call(
        _mm_body,
        grid=(grid,),
        in_specs=[
            pl.BlockSpec((block_rows, k), lambda i: (i, 0)),
            pl.BlockSpec((k, n), lambda i: (0, 0)),
            pl.BlockSpec((1, n), lambda i: (0, 0)),
        ],
        out_specs=pl.BlockSpec((block_rows, n), lambda i: (i, 0)),
        out_shape=jax.ShapeDtypeStruct((m, n), jnp.float32),
    )(x, w, b.reshape(1, n))


def _att_mat(att):
    a = jnp.zeros((_H * _C, _H), jnp.float32)
    a = a.at[: _C, 0].set(att[0]).at[_C :, 1].set(att[1])
    return a


def _edge_pass(hs, a_s, a_d, src, dst, n_dst):
    e = a_s[src] + a_d[dst]
    e = jnp.where(e >= 0, e, 0.2 * e)
    ex = jnp.exp(e)
    den = jax.ops.segment_sum(ex, dst, num_segments=n_dst)
    msg = hs[src].reshape(-1, _H, _C) * ex[:, :, None]
    num = jax.ops.segment_sum(msg.reshape(-1, _H * _C), dst, num_segments=n_dst)
    return num, den


def _conv(xs, xd, ei, p):
    a_src = _att_mat(p['att_src'])
    a_dst = _att_mat(p['att_dst'])
    w_cat = jnp.concatenate([p['W_src'], p['W_src'] @ a_src], axis=1)
    hs_as = _mm(xs, w_cat, jnp.zeros((_H * _C + _H,), jnp.float32))
    hs = hs_as[:, : _H * _C]
    a_s = hs_as[:, _H * _C :]
    a_d = _mm(xd, p['W_dst'] @ a_dst, jnp.zeros((_H,), jnp.float32))
    num, den = _edge_pass(hs, a_s, a_d, ei[0], ei[1], xd.shape[0])
    den = jnp.repeat(den + 1e-16, _C, axis=1)
    return num / den + p['bias']


def kernel(x_user, x_tweet, params, edge_index_ut, edge_index_tu):
    xu = _mm(x_user, params['W_emb_user'], params['b_emb_user'])
    xt = _mm(x_tweet, params['W_emb_tweet'], params['b_emb_tweet'])
    nt = _conv(xu, xt, edge_index_ut, params['conv0_ut'])
    nu = _conv(xt, xu, edge_index_tu, params['conv0_tu'])
    xt = jax.nn.elu(nt)
    xu = jax.nn.elu(nu)
    nt = _conv(xu, xt, edge_index_ut, params['conv1_ut'])
    xt = jax.nn.elu(nt)
    return _mm(xt, params['W_out'], params['b_out'])


# speedup vs baseline: 73.8042x; 7.1284x over previous
import functools

import jax
import jax.numpy as jnp
from jax import lax
from jax.experimental import pallas as pl
from jax.experimental.pallas import tpu as pltpu
from jax.experimental.pallas import tpu_sc as plsc

_N = 50000
_E = 600000
_H = 2
_C = 32
_HID = _H * _C

_DW = 2
_CH = 128
_NSUB = 16
_CPT = 294
_EPAD = _NSUB * _CPT * _CH
_HALF = _N // 2
_RPT = 1568
_ACCR = _NSUB * _RPT


def _prep_body(x_ref, w_ref, b_ref, v_ref, ab_ref, t_ref, a_ref):
    x = x_ref[...]
    t_ref[...] = (
        jnp.dot(x, w_ref[...], preferred_element_type=jnp.float32)
        + b_ref[...]
    )
    a_ref[...] = (
        jnp.dot(x, v_ref[...], preferred_element_type=jnp.float32)
        + ab_ref[...]
    )


def _prep(x, w, b, v, ab, block_rows=5000):
    m, k = x.shape
    return pl.pallas_call(
        _prep_body,
        grid=(m // block_rows,),
        in_specs=[
            pl.BlockSpec((block_rows, k), lambda i: (i, 0)),
            pl.BlockSpec((k, _HID), lambda i: (0, 0)),
            pl.BlockSpec((1, _HID), lambda i: (0, 0)),
            pl.BlockSpec((k, 8), lambda i: (0, 0)),
            pl.BlockSpec((1, 8), lambda i: (0, 0)),
        ],
        out_specs=[
            pl.BlockSpec((block_rows, _HID), lambda i: (i, 0)),
            pl.BlockSpec((block_rows, 8), lambda i: (i, 0)),
        ],
        out_shape=[
            jax.ShapeDtypeStruct((m, _HID), jnp.float32),
            jax.ShapeDtypeStruct((m, 8), jnp.float32),
        ],
    )(x, w, b.reshape(1, _HID), v, ab.reshape(1, 8))


def _elu_bank(num_ref, den_ref, bias_ref):
    num = num_ref[...]
    n_rows = num.shape[0]
    d0 = den_ref[:, 0:1] + 1e-16
    d1 = den_ref[:, 1:2] + 1e-16
    den = jnp.concatenate(
        [jnp.broadcast_to(d0, (n_rows, _C)),
         jnp.broadcast_to(d1, (n_rows, _C))], axis=1)
    x = num / den + bias_ref[...]
    return jnp.where(x > 0, x, jnp.exp(jnp.minimum(x, 0.0)) - 1.0)


def _epi_prep_body(num_ref, den_ref, bias_ref, w_ref, v_ref, t_ref, a_ref):
    x = _elu_bank(num_ref, den_ref, bias_ref)
    t_ref[...] = jnp.dot(x, w_ref[...], preferred_element_type=jnp.float32)
    a_ref[...] = jnp.dot(x, v_ref[...], preferred_element_type=jnp.float32)


def _epi_prep(num, den, bias, w, v, block_rows=5000):
    m = num.shape[0]
    return pl.pallas_call(
        _epi_prep_body,
        grid=(m // block_rows,),
        in_specs=[
            pl.BlockSpec((block_rows, _HID), lambda i: (i, 0)),
            pl.BlockSpec((block_rows, _DW), lambda i: (i, 0)),
            pl.BlockSpec((1, _HID), lambda i: (0, 0)),
            pl.BlockSpec((_HID, _HID), lambda i: (0, 0)),
            pl.BlockSpec((_HID, 8), lambda i: (0, 0)),
        ],
        out_specs=[
            pl.BlockSpec((block_rows, _HID), lambda i: (i, 0)),
            pl.BlockSpec((block_rows, 8), lambda i: (i, 0)),
        ],
        out_shape=[
            jax.ShapeDtypeStruct((m, _HID), jnp.float32),
            jax.ShapeDtypeStruct((m, 8), jnp.float32),
        ],
    )(num, den, bias.reshape(1, _HID), w, v)


def _epi_mm_body(num_ref, den_ref, bias_ref, w_ref, b_ref, o_ref):
    x = _elu_bank(num_ref, den_ref, bias_ref)
    o_ref[...] = (
        jnp.dot(x, w_ref[...], preferred_element_type=jnp.float32)
        + b_ref[...]
    )


def _epi_mm(num, den, bias, w, b, block_rows=5000):
    m = num.shape[0]
    n = w.shape[1]
    return pl.pallas_call(
        _epi_mm_body,
        grid=(m // block_rows,),
        in_specs=[
            pl.BlockSpec((block_rows, _HID), lambda i: (i, 0)),
            pl.BlockSpec((block_rows, _DW), lambda i: (i, 0)),
            pl.BlockSpec((1, _HID), lambda i: (0, 0)),
            pl.BlockSpec((_HID, n), lambda i: (0, 0)),
            pl.BlockSpec((1, n), lambda i: (0, 0)),
        ],
        out_specs=pl.BlockSpec((block_rows, n), lambda i: (i, 0)),
        out_shape=jax.ShapeDtypeStruct((m, n), jnp.float32),
    )(num, den, bias.reshape(1, _HID), w, b.reshape(1, n))


def _edge_body(table, as0, as1, ad0, ad1, src, dst, znum, zden,
               num_out, den_out,
               acc, accd, srcbuf, dstbuf, lidxbuf,
               as0b, as1b, ad0b, ad1b, exbuf, exidx, msgbuf, sem_g, sem_a):
    c = lax.axis_index("c")
    s = lax.axis_index("s")
    chalf = c * _HALF

    pltpu.sync_copy(znum, acc.at[pl.ds(s * _RPT, _RPT)])
    pltpu.sync_copy(zden, accd.at[pl.ds(s * 2 * _RPT, 2 * _RPT)])
    plsc.subcore_barrier()

    iota = lax.iota(jnp.int32, 16)

    def chunk(i, _):
        base = pl.multiple_of((s * _CPT + i) * _CH, _CH)
        pltpu.sync_copy(src.at[pl.ds(base, _CH)], srcbuf)
        pltpu.sync_copy(dst.at[pl.ds(base, _CH)], dstbuf)
        cp_g = pltpu.async_copy(table.at[srcbuf], msgbuf, sem_g)
        cps = [
            pltpu.async_copy(as0.at[srcbuf], as0b, sem_a),
            pltpu.async_copy(as1.at[srcbuf], as1b, sem_a),
            pltpu.async_copy(ad0.at[dstbuf], ad0b, sem_a),
            pltpu.async_copy(ad1.at[dstbuf], ad1b, sem_a),
        ]
        cp_g.wait()
        for cp in cps:
            cp.wait()
        for v in range(8):
            sl = pl.ds(v * 16, 16)
            dv = dstbuf[sl]
            local = dv - chalf
            okv = (local >= 0) & (local < _HALF) & ((base + v * 16 + iota) < _E)
            local = jnp.where(local < 0, local + _HALF, local)
            local = jnp.where(local >= _HALF, local - _HALF, local)
            lidxbuf[sl] = local
            exidx[0, sl] = local
            exidx[1, sl] = local + _ACCR
            e0 = as0b[sl] + ad0b[sl]
            e0 = jnp.where(e0 >= 0.0, e0, e0 * 0.2)
            exbuf[0, sl] = jnp.where(okv, jnp.exp(e0), 0.0)
            e1 = as1b[sl] + ad1b[sl]
            e1 = jnp.where(e1 >= 0.0, e1, e1 * 0.2)
            exbuf[1, sl] = jnp.where(okv, jnp.exp(e1), 0.0)
        for g in range(8):
            e0v = exbuf[0, pl.ds(g * 16, 16)]
            e1v = exbuf[1, pl.ds(g * 16, 16)]
            for t in range(16):
                k = g * 16 + t
                x0 = jnp.full((16,), e0v[t], jnp.float32)
                x1 = jnp.full((16,), e1v[t], jnp.float32)
                for j in range(4):
                    sl2 = pl.ds(j * 16, 16)
                    msgbuf[k, sl2] = msgbuf[k, sl2] * (x0 if j < 2 else x1)
        pltpu.sync_copy(msgbuf, acc.at[lidxbuf], add=True)
        pltpu.sync_copy(exbuf.at[0], accd.at[exidx.at[0]], add=True)
        pltpu.sync_copy(exbuf.at[1], accd.at[exidx.at[1]], add=True)
        return ()

    lax.fori_loop(0, _CPT, chunk, (), unroll=False)

    plsc.subcore_barrier()
    rbase = s * _RPT
    pltpu.sync_copy(acc.at[pl.ds(rbase, _RPT)],
                    num_out.at[pl.ds(c * _ACCR + rbase, _RPT)])
    for h in range(2):
        pltpu.sync_copy(
            accd.at[pl.ds(h * _ACCR + rbase, _RPT)],
            den_out.at[pl.ds((c * 2 + h) * _ACCR + rbase, _RPT)])


@functools.partial(
    pl.kernel,
    out_type=[
        jax.ShapeDtypeStruct((2 * _ACCR, _HID), jnp.float32),
        jax.ShapeDtypeStruct((4 * _ACCR,), jnp.float32),
    ],
    mesh=plsc.VectorSubcoreMesh(core_axis_name="c", subcore_axis_name="s"),
    scratch_types=[
        pltpu.VMEM_SHARED((_ACCR, _HID), jnp.float32),
        pltpu.VMEM_SHARED((2 * _ACCR,), jnp.float32),
        pltpu.VMEM((_CH,), jnp.int32),
        pltpu.VMEM((_CH,), jnp.int32),
        pltpu.VMEM((_CH,), jnp.int32),
        pltpu.VMEM((_CH,), jnp.float32),
        pltpu.VMEM((_CH,), jnp.float32),
        pltpu.VMEM((_CH,), jnp.float32),
        pltpu.VMEM((_CH,), jnp.float32),
        pltpu.VMEM((2, _CH), jnp.float32),
        pltpu.VMEM((2, _CH), jnp.int32),
        pltpu.VMEM((_CH, _HID), jnp.float32),
        pltpu.SemaphoreType.DMA,
        pltpu.SemaphoreType.DMA,
    ],
    compiler_params=pltpu.CompilerParams(use_tc_tiling_on_sc=False),
)
def _edge_pass(*refs):
    _edge_body(*refs)


def _att_mat(att):
    a = jnp.zeros((_HID, _H), jnp.float32)
    return a.at[:_C, 0].set(att[0]).at[_C:, 1].set(att[1])


def _v_ext(p_src, p_dst):
    vs = p_src['W_src'] @ _att_mat(p_src['att_src'])
    vd = p_dst['W_dst'] @ _att_mat(p_dst['att_dst'])
    return jnp.concatenate([vs, vd, jnp.zeros((_HID, 4), jnp.float32)], axis=1)


def _pad_edges(ei):
    pad = jnp.zeros((_EPAD - _E,), jnp.int32)
    return (jnp.concatenate([ei[0], pad]), jnp.concatenate([ei[1], pad]))


def _unbank(num_raw, den_raw):
    num = jnp.concatenate([num_raw[:_HALF], num_raw[_ACCR:_ACCR + _HALF]])
    dr = den_raw.reshape(2, 2, _ACCR)
    den = jnp.concatenate([dr[0, :, :_HALF], dr[1, :, :_HALF]], axis=1).T
    return num, den


def kernel(x_user, x_tweet, params, edge_index_ut, edge_index_tu):
    p = params
    su, du = _pad_edges(edge_index_ut)
    st, dt = _pad_edges(edge_index_tu)
    znum = jnp.zeros((_RPT, _HID), jnp.float32)
    zden = jnp.zeros((2 * _RPT,), jnp.float32)
    c0u, c0t, c1u = p['conv0_ut'], p['conv0_tu'], p['conv1_ut']

    tab_u, a_u = _prep(x_user, p['W_emb_user'] @ c0u['W_src'],
                       p['b_emb_user'] @ c0u['W_src'],
                       p['W_emb_user'] @ _v_ext(c0u, c0t),
                       p['b_emb_user'] @ _v_ext(c0u, c0t))
    tab_t, a_t = _prep(x_tweet, p['W_emb_tweet'] @ c0t['W_src'],
                       p['b_emb_tweet'] @ c0t['W_src'],
                       p['W_emb_tweet'] @ _v_ext(c0t, c0u),
                       p['b_emb_tweet'] @ _v_ext(c0t, c0u))
    n_ut0, d_ut0 = _unbank(*_edge_pass(tab_u, a_u[:, 0], a_u[:, 1],
                                       a_t[:, 2], a_t[:, 3], su, du,
                                       znum, zden))
    n_tu0, d_tu0 = _unbank(*_edge_pass(tab_t, a_t[:, 0], a_t[:, 1],
                                       a_u[:, 2], a_u[:, 3], st, dt,
                                       znum, zden))

    tab1, a1s = _epi_prep(n_tu0, d_tu0, c0t['bias'], c1u['W_src'],
                          _v_ext(c1u, c1u))
    _, a1d = _epi_prep(n_ut0, d_ut0, c0u['bias'], c1u['W_src'],
                       _v_ext(c1u, c1u))
    n_ut1, d_ut1 = _unbank(*_edge_pass(tab1, a1s[:, 0], a1s[:, 1],
                                       a1d[:, 2], a1d[:, 3], su, du,
                                       znum, zden))

    return _epi_mm(n_ut1, d_ut1, c1u['bias'], p['W_out'], p['b_out'])


# speedup vs baseline: 120.6543x; 1.6348x over previous
import functools

import jax
import jax.numpy as jnp
from jax import lax
from jax.experimental import pallas as pl
from jax.experimental.pallas import tpu as pltpu
from jax.experimental.pallas import tpu_sc as plsc

_N = 50000
_E = 600000
_H = 2
_C = 32
_HID = _H * _C

_DW = 2
_CH = 128
_NSUB = 16
_CPT = 294
_EPAD = _NSUB * _CPT * _CH
_HALF = _N // 2
_RPT = 1568
_ACCR = _NSUB * _RPT


def _prep_body(x_ref, w_ref, b_ref, v_ref, ab_ref, t_ref, a_ref):
    x = x_ref[...]
    t_ref[...] = (
        jnp.dot(x, w_ref[...], preferred_element_type=jnp.float32)
        + b_ref[...]
    )
    a_ref[...] = (
        jnp.dot(x, v_ref[...], preferred_element_type=jnp.float32)
        + ab_ref[...]
    )


def _prep(x, w, b, v, ab, block_rows=5000):
    m, k = x.shape
    return pl.pallas_call(
        _prep_body,
        grid=(m // block_rows,),
        in_specs=[
            pl.BlockSpec((block_rows, k), lambda i: (i, 0)),
            pl.BlockSpec((k, _HID), lambda i: (0, 0)),
            pl.BlockSpec((1, _HID), lambda i: (0, 0)),
            pl.BlockSpec((k, 8), lambda i: (0, 0)),
            pl.BlockSpec((1, 8), lambda i: (0, 0)),
        ],
        out_specs=[
            pl.BlockSpec((block_rows, _HID), lambda i: (i, 0)),
            pl.BlockSpec((block_rows, 8), lambda i: (i, 0)),
        ],
        out_shape=[
            jax.ShapeDtypeStruct((m, _HID), jnp.float32),
            jax.ShapeDtypeStruct((m, 8), jnp.float32),
        ],
    )(x, w, b.reshape(1, _HID), v, ab.reshape(1, 8))


def _elu_bank(num_ref, den_ref, bias_ref):
    num = num_ref[...]
    n_rows = num.shape[0]
    d0 = den_ref[:, 0:1] + 1e-16
    d1 = den_ref[:, 1:2] + 1e-16
    den = jnp.concatenate(
        [jnp.broadcast_to(d0, (n_rows, _C)),
         jnp.broadcast_to(d1, (n_rows, _C))], axis=1)
    x = num / den + bias_ref[...]
    return jnp.where(x > 0, x, jnp.exp(jnp.minimum(x, 0.0)) - 1.0)


def _epi_prep_body(num_ref, den_ref, bias_ref, w_ref, v_ref, t_ref, a_ref):
    x = _elu_bank(num_ref, den_ref, bias_ref)
    t_ref[...] = jnp.dot(x, w_ref[...], preferred_element_type=jnp.float32)
    a_ref[...] = jnp.dot(x, v_ref[...], preferred_element_type=jnp.float32)


def _epi_prep(num, den, bias, w, v, block_rows=5000):
    m = num.shape[0]
    return pl.pallas_call(
        _epi_prep_body,
        grid=(m // block_rows,),
        in_specs=[
            pl.BlockSpec((block_rows, _HID), lambda i: (i, 0)),
            pl.BlockSpec((block_rows, _DW), lambda i: (i, 0)),
            pl.BlockSpec((1, _HID), lambda i: (0, 0)),
            pl.BlockSpec((_HID, _HID), lambda i: (0, 0)),
            pl.BlockSpec((_HID, 8), lambda i: (0, 0)),
        ],
        out_specs=[
            pl.BlockSpec((block_rows, _HID), lambda i: (i, 0)),
            pl.BlockSpec((block_rows, 8), lambda i: (i, 0)),
        ],
        out_shape=[
            jax.ShapeDtypeStruct((m, _HID), jnp.float32),
            jax.ShapeDtypeStruct((m, 8), jnp.float32),
        ],
    )(num, den, bias.reshape(1, _HID), w, v)


def _epi_mm_body(num_ref, den_ref, bias_ref, w_ref, b_ref, o_ref):
    x = _elu_bank(num_ref, den_ref, bias_ref)
    o_ref[...] = (
        jnp.dot(x, w_ref[...], preferred_element_type=jnp.float32)
        + b_ref[...]
    )


def _epi_mm(num, den, bias, w, b, block_rows=5000):
    m = num.shape[0]
    n = w.shape[1]
    return pl.pallas_call(
        _epi_mm_body,
        grid=(m // block_rows,),
        in_specs=[
            pl.BlockSpec((block_rows, _HID), lambda i: (i, 0)),
            pl.BlockSpec((block_rows, _DW), lambda i: (i, 0)),
            pl.BlockSpec((1, _HID), lambda i: (0, 0)),
            pl.BlockSpec((_HID, n), lambda i: (0, 0)),
            pl.BlockSpec((1, n), lambda i: (0, 0)),
        ],
        out_specs=pl.BlockSpec((block_rows, n), lambda i: (i, 0)),
        out_shape=jax.ShapeDtypeStruct((m, n), jnp.float32),
    )(num, den, bias.reshape(1, _HID), w, b.reshape(1, n))


def _edge_body(table, as0, as1, ad0, ad1, src, dst, znum, zden,
               num_out, den_out,
               acc, accd, srcbuf, dstbuf, lidxbuf,
               as0b, as1b, ad0b, ad1b, exbuf, exidx, msgbuf,
               sem_i0, sem_i1, sem_g0, sem_g1, sem_a0, sem_a1):
    c = lax.axis_index("c")
    s = lax.axis_index("s")
    chalf = c * _HALF
    sem_i = (sem_i0, sem_i1)
    sem_g = (sem_g0, sem_g1)
    sem_a = (sem_a0, sem_a1)

    pltpu.sync_copy(znum, acc.at[pl.ds(s * _RPT, _RPT)])
    pltpu.sync_copy(zden, accd.at[pl.ds(s * 2 * _RPT, 2 * _RPT)])
    plsc.subcore_barrier()

    iota = lax.iota(jnp.int32, 16)

    def issue_idx(i, b):
        base = pl.multiple_of((s * _CPT + i) * _CH, _CH)
        pltpu.async_copy(src.at[pl.ds(base, _CH)], srcbuf.at[b], sem_i[b])
        pltpu.async_copy(dst.at[pl.ds(base, _CH)], dstbuf.at[b], sem_i[b])

    def drain_idx(b):
        pltpu.make_async_copy(src.at[pl.ds(0, _CH)], srcbuf.at[b],
                              sem_i[b]).wait()
        pltpu.make_async_copy(dst.at[pl.ds(0, _CH)], dstbuf.at[b],
                              sem_i[b]).wait()

    def issue_gathers(b):
        pltpu.async_copy(table.at[srcbuf.at[b]], msgbuf.at[b], sem_g[b])
        pltpu.async_copy(as0.at[srcbuf.at[b]], as0b.at[b], sem_a[b])
        pltpu.async_copy(as1.at[srcbuf.at[b]], as1b.at[b], sem_a[b])
        pltpu.async_copy(ad0.at[dstbuf.at[b]], ad0b.at[b], sem_a[b])
        pltpu.async_copy(ad1.at[dstbuf.at[b]], ad1b.at[b], sem_a[b])

    def drain_gathers(b):
        pltpu.make_async_copy(table.at[pl.ds(0, _CH)], msgbuf.at[b],
                              sem_g[b]).wait()
        for ref, buf in ((as0, as0b), (as1, as1b), (ad0, ad0b), (ad1, ad1b)):
            pltpu.make_async_copy(ref.at[pl.ds(0, _CH)], buf.at[b],
                                  sem_a[b]).wait()

    def compute(i, b):
        base = pl.multiple_of((s * _CPT + i) * _CH, _CH)
        for v in range(8):
            sl = pl.ds(v * 16, 16)
            dv = dstbuf[b, sl]
            local = dv - chalf
            okv = (local >= 0) & (local < _HALF) & ((base + v * 16 + iota) < _E)
            local = jnp.where(local < 0, local + _HALF, local)
            local = jnp.where(local >= _HALF, local - _HALF, local)
            lidxbuf[sl] = local
            exidx[0, sl] = local
            exidx[1, sl] = local + _ACCR
            e0 = as0b[b, sl] + ad0b[b, sl]
            e0 = jnp.where(e0 >= 0.0, e0, e0 * 0.2)
            exbuf[0, sl] = jnp.where(okv, jnp.exp(e0), 0.0)
            e1 = as1b[b, sl] + ad1b[b, sl]
            e1 = jnp.where(e1 >= 0.0, e1, e1 * 0.2)
            exbuf[1, sl] = jnp.where(okv, jnp.exp(e1), 0.0)
        for g in range(8):
            e0v = exbuf[0, pl.ds(g * 16, 16)]
            e1v = exbuf[1, pl.ds(g * 16, 16)]
            for t in range(16):
                k = g * 16 + t
                x0 = jnp.full((16,), e0v[t], jnp.float32)
                x1 = jnp.full((16,), e1v[t], jnp.float32)
                for j in range(4):
                    sl2 = pl.ds(j * 16, 16)
                    msgbuf[b, k, sl2] = msgbuf[b, k, sl2] * (x0 if j < 2 else x1)
        pltpu.sync_copy(msgbuf.at[b], acc.at[lidxbuf], add=True)
        pltpu.sync_copy(exbuf.at[0], accd.at[exidx.at[0]], add=True)
        pltpu.sync_copy(exbuf.at[1], accd.at[exidx.at[1]], add=True)

    issue_idx(0, 0)
    issue_idx(1, 1)
    drain_idx(0)
    issue_gathers(0)

    def body2(m, _):
        for b in (0, 1):
            i = 2 * m + b
            drain_gathers(b)

            @pl.when(i + 1 < _CPT)
            def _():
                drain_idx(1 - b)
                issue_gathers(1 - b)

            compute(i, b)

            @pl.when(i + 2 < _CPT)
            def _():
                issue_idx(i + 2, b)
        return ()

    lax.fori_loop(0, _CPT // 2, body2, (), unroll=False)

    plsc.subcore_barrier()
    rbase = s * _RPT
    pltpu.sync_copy(acc.at[pl.ds(rbase, _RPT)],
                    num_out.at[pl.ds(c * _ACCR + rbase, _RPT)])
    for h in range(2):
        pltpu.sync_copy(
            accd.at[pl.ds(h * _ACCR + rbase, _RPT)],
            den_out.at[pl.ds((c * 2 + h) * _ACCR + rbase, _RPT)])


@functools.partial(
    pl.kernel,
    out_type=[
        jax.ShapeDtypeStruct((2 * _ACCR, _HID), jnp.float32),
        jax.ShapeDtypeStruct((4 * _ACCR,), jnp.float32),
    ],
    mesh=plsc.VectorSubcoreMesh(core_axis_name="c", subcore_axis_name="s"),
    scratch_types=[
        pltpu.VMEM_SHARED((_ACCR, _HID), jnp.float32),
        pltpu.VMEM_SHARED((2 * _ACCR,), jnp.float32),
        pltpu.VMEM((2, _CH), jnp.int32),
        pltpu.VMEM((2, _CH), jnp.int32),
        pltpu.VMEM((_CH,), jnp.int32),
        pltpu.VMEM((2, _CH), jnp.float32),
        pltpu.VMEM((2, _CH), jnp.float32),
        pltpu.VMEM((2, _CH), jnp.float32),
        pltpu.VMEM((2, _CH), jnp.float32),
        pltpu.VMEM((2, _CH), jnp.float32),
        pltpu.VMEM((2, _CH), jnp.int32),
        pltpu.VMEM((2, _CH, _HID), jnp.float32),
        pltpu.SemaphoreType.DMA,
        pltpu.SemaphoreType.DMA,
        pltpu.SemaphoreType.DMA,
        pltpu.SemaphoreType.DMA,
        pltpu.SemaphoreType.DMA,
        pltpu.SemaphoreType.DMA,
    ],
    compiler_params=pltpu.CompilerParams(use_tc_tiling_on_sc=False),
)
def _edge_pass(*refs):
    _edge_body(*refs)


def _att_mat(att):
    a = jnp.zeros((_HID, _H), jnp.float32)
    return a.at[:_C, 0].set(att[0]).at[_C:, 1].set(att[1])


def _v_ext(p_src, p_dst):
    vs = p_src['W_src'] @ _att_mat(p_src['att_src'])
    vd = p_dst['W_dst'] @ _att_mat(p_dst['att_dst'])
    return jnp.concatenate([vs, vd, jnp.zeros((_HID, 4), jnp.float32)], axis=1)


def _pad_edges(ei):
    pad = jnp.zeros((_EPAD - _E,), jnp.int32)
    return (jnp.concatenate([ei[0], pad]), jnp.concatenate([ei[1], pad]))


def _unbank(num_raw, den_raw):
    num = jnp.concatenate([num_raw[:_HALF], num_raw[_ACCR:_ACCR + _HALF]])
    dr = den_raw.reshape(2, 2, _ACCR)
    den = jnp.concatenate([dr[0, :, :_HALF], dr[1, :, :_HALF]], axis=1).T
    return num, den


def kernel(x_user, x_tweet, params, edge_index_ut, edge_index_tu):
    p = params
    su, du = _pad_edges(edge_index_ut)
    st, dt = _pad_edges(edge_index_tu)
    znum = jnp.zeros((_RPT, _HID), jnp.float32)
    zden = jnp.zeros((2 * _RPT,), jnp.float32)
    c0u, c0t, c1u = p['conv0_ut'], p['conv0_tu'], p['conv1_ut']

    tab_u, a_u = _prep(x_user, p['W_emb_user'] @ c0u['W_src'],
                       p['b_emb_user'] @ c0u['W_src'],
                       p['W_emb_user'] @ _v_ext(c0u, c0t),
                       p['b_emb_user'] @ _v_ext(c0u, c0t))
    tab_t, a_t = _prep(x_tweet, p['W_emb_tweet'] @ c0t['W_src'],
                       p['b_emb_tweet'] @ c0t['W_src'],
                       p['W_emb_tweet'] @ _v_ext(c0t, c0u),
                       p['b_emb_tweet'] @ _v_ext(c0t, c0u))
    n_ut0, d_ut0 = _unbank(*_edge_pass(tab_u, a_u[:, 0], a_u[:, 1],
                                       a_t[:, 2], a_t[:, 3], su, du,
                                       znum, zden))
    n_tu0, d_tu0 = _unbank(*_edge_pass(tab_t, a_t[:, 0], a_t[:, 1],
                                       a_u[:, 2], a_u[:, 3], st, dt,
                                       znum, zden))

    tab1, a1s = _epi_prep(n_tu0, d_tu0, c0t['bias'], c1u['W_src'],
                          _v_ext(c1u, c1u))
    _, a1d = _epi_prep(n_ut0, d_ut0, c0u['bias'], c1u['W_src'],
                       _v_ext(c1u, c1u))
    n_ut1, d_ut1 = _unbank(*_edge_pass(tab1, a1s[:, 0], a1s[:, 1],
                                       a1d[:, 2], a1d[:, 3], su, du,
                                       znum, zden))

    return _epi_mm(n_ut1, d_ut1, c1u['bias'], p['W_out'], p['b_out'])


# speedup vs baseline: 124.8962x; 1.0352x over previous
import functools

import jax
import jax.numpy as jnp
from jax import lax
from jax.experimental import pallas as pl
from jax.experimental.pallas import tpu as pltpu
from jax.experimental.pallas import tpu_sc as plsc

_N = 50000
_E = 600000
_H = 2
_C = 32
_HID = _H * _C

_DW = 2
_CH = 128
_NSUB = 16
_CPT = 294
_EPAD = _NSUB * _CPT * _CH
_HALF = _N // 2
_RPT = 1568
_ACCR = _NSUB * _RPT


def _prep_body(x_ref, w_ref, b_ref, v_ref, ab_ref, t_ref, a_ref):
    x = x_ref[...]
    t_ref[...] = (
        jnp.dot(x, w_ref[...], preferred_element_type=jnp.float32)
        + b_ref[...]
    )
    a_ref[...] = (
        jnp.dot(x, v_ref[...], preferred_element_type=jnp.float32)
        + ab_ref[...]
    )


def _prep(x, w, b, v, ab, block_rows=5000):
    m, k = x.shape
    return pl.pallas_call(
        _prep_body,
        grid=(m // block_rows,),
        in_specs=[
            pl.BlockSpec((block_rows, k), lambda i: (i, 0)),
            pl.BlockSpec((k, _HID), lambda i: (0, 0)),
            pl.BlockSpec((1, _HID), lambda i: (0, 0)),
            pl.BlockSpec((k, 8), lambda i: (0, 0)),
            pl.BlockSpec((1, 8), lambda i: (0, 0)),
        ],
        out_specs=[
            pl.BlockSpec((block_rows, _HID), lambda i: (i, 0)),
            pl.BlockSpec((block_rows, 8), lambda i: (i, 0)),
        ],
        out_shape=[
            jax.ShapeDtypeStruct((m, _HID), jnp.float32),
            jax.ShapeDtypeStruct((m, 8), jnp.float32),
        ],
    )(x, w, b.reshape(1, _HID), v, ab.reshape(1, 8))


def _elu_bank(num_ref, den_ref, bias_ref):
    num = num_ref[...]
    n_rows = num.shape[0]
    d0 = den_ref[:, 0:1] + 1e-16
    d1 = den_ref[:, 1:2] + 1e-16
    den = jnp.concatenate(
        [jnp.broadcast_to(d0, (n_rows, _C)),
         jnp.broadcast_to(d1, (n_rows, _C))], axis=1)
    x = num / den + bias_ref[...]
    return jnp.where(x > 0, x, jnp.exp(jnp.minimum(x, 0.0)) - 1.0)


def _epi_prep_body(num_ref, den_ref, bias_ref, w_ref, v_ref, t_ref, a_ref):
    x = _elu_bank(num_ref, den_ref, bias_ref)
    t_ref[...] = jnp.dot(x, w_ref[...], preferred_element_type=jnp.float32)
    a_ref[...] = jnp.dot(x, v_ref[...], preferred_element_type=jnp.float32)


def _epi_prep(num, den, bias, w, v, block_rows=5000):
    m = num.shape[0]
    return pl.pallas_call(
        _epi_prep_body,
        grid=(m // block_rows,),
        in_specs=[
            pl.BlockSpec((block_rows, _HID), lambda i: (i, 0)),
            pl.BlockSpec((block_rows, _DW), lambda i: (i, 0)),
            pl.BlockSpec((1, _HID), lambda i: (0, 0)),
            pl.BlockSpec((_HID, _HID), lambda i: (0, 0)),
            pl.BlockSpec((_HID, 8), lambda i: (0, 0)),
        ],
        out_specs=[
            pl.BlockSpec((block_rows, _HID), lambda i: (i, 0)),
            pl.BlockSpec((block_rows, 8), lambda i: (i, 0)),
        ],
        out_shape=[
            jax.ShapeDtypeStruct((m, _HID), jnp.float32),
            jax.ShapeDtypeStruct((m, 8), jnp.float32),
        ],
    )(num, den, bias.reshape(1, _HID), w, v)


def _epi_mm_body(num_ref, den_ref, bias_ref, w_ref, b_ref, o_ref):
    x = _elu_bank(num_ref, den_ref, bias_ref)
    o_ref[...] = (
        jnp.dot(x, w_ref[...], preferred_element_type=jnp.float32)
        + b_ref[...]
    )


def _epi_mm(num, den, bias, w, b, block_rows=5000):
    m = num.shape[0]
    n = w.shape[1]
    return pl.pallas_call(
        _epi_mm_body,
        grid=(m // block_rows,),
        in_specs=[
            pl.BlockSpec((block_rows, _HID), lambda i: (i, 0)),
            pl.BlockSpec((block_rows, _DW), lambda i: (i, 0)),
            pl.BlockSpec((1, _HID), lambda i: (0, 0)),
            pl.BlockSpec((_HID, n), lambda i: (0, 0)),
            pl.BlockSpec((1, n), lambda i: (0, 0)),
        ],
        out_specs=pl.BlockSpec((block_rows, n), lambda i: (i, 0)),
        out_shape=jax.ShapeDtypeStruct((m, n), jnp.float32),
    )(num, den, bias.reshape(1, _HID), w, b.reshape(1, n))


def _edge_body(table, as0, as1, ad0, ad1, src, dst, znum, zden,
               num_out, den_out,
               acc, accd, srcbuf, dstbuf, lidxbuf,
               as0b, as1b, ad0b, ad1b, exbuf, exidx, msgbuf,
               sem_i0, sem_i1, sem_g0, sem_g1, sem_a0, sem_a1,
               sem_s0, sem_s1):
    c = lax.axis_index("c")
    s = lax.axis_index("s")
    chalf = c * _HALF
    sem_i = (sem_i0, sem_i1)
    sem_g = (sem_g0, sem_g1)
    sem_a = (sem_a0, sem_a1)
    sem_s = (sem_s0, sem_s1)

    pltpu.sync_copy(znum, acc.at[pl.ds(s * _RPT, _RPT)])
    pltpu.sync_copy(zden, accd.at[pl.ds(s * 2 * _RPT, 2 * _RPT)])
    plsc.subcore_barrier()

    iota = lax.iota(jnp.int32, 16)

    def issue_idx(i, b):
        base = pl.multiple_of((s * _CPT + i) * _CH, _CH)
        pltpu.async_copy(src.at[pl.ds(base, _CH)], srcbuf.at[b], sem_i[b])
        pltpu.async_copy(dst.at[pl.ds(base, _CH)], dstbuf.at[b], sem_i[b])

    def drain_idx(b):
        pltpu.make_async_copy(src.at[pl.ds(0, _CH)], srcbuf.at[b],
                              sem_i[b]).wait()
        pltpu.make_async_copy(dst.at[pl.ds(0, _CH)], dstbuf.at[b],
                              sem_i[b]).wait()

    def issue_gathers(b):
        pltpu.async_copy(table.at[srcbuf.at[b]], msgbuf.at[b], sem_g[b])
        pltpu.async_copy(as0.at[srcbuf.at[b]], as0b.at[b], sem_a[b])
        pltpu.async_copy(as1.at[srcbuf.at[b]], as1b.at[b], sem_a[b])
        pltpu.async_copy(ad0.at[dstbuf.at[b]], ad0b.at[b], sem_a[b])
        pltpu.async_copy(ad1.at[dstbuf.at[b]], ad1b.at[b], sem_a[b])

    def drain_gathers(b):
        pltpu.make_async_copy(table.at[pl.ds(0, _CH)], msgbuf.at[b],
                              sem_g[b]).wait()
        for ref, buf in ((as0, as0b), (as1, as1b), (ad0, ad0b), (ad1, ad1b)):
            pltpu.make_async_copy(ref.at[pl.ds(0, _CH)], buf.at[b],
                                  sem_a[b]).wait()

    def compute(i, b):
        base = pl.multiple_of((s * _CPT + i) * _CH, _CH)
        for v in range(8):
            sl = pl.ds(v * 16, 16)
            dv = dstbuf[b, sl]
            local = dv - chalf
            okv = (local >= 0) & (local < _HALF) & ((base + v * 16 + iota) < _E)
            local = jnp.where(local < 0, local + _HALF, local)
            local = jnp.where(local >= _HALF, local - _HALF, local)
            lidxbuf[b, sl] = local
            exidx[b, 0, sl] = local
            exidx[b, 1, sl] = local + _ACCR
            e0 = as0b[b, sl] + ad0b[b, sl]
            e0 = jnp.where(e0 >= 0.0, e0, e0 * 0.2)
            exbuf[b, 0, sl] = jnp.where(okv, jnp.exp(e0), 0.0)
            e1 = as1b[b, sl] + ad1b[b, sl]
            e1 = jnp.where(e1 >= 0.0, e1, e1 * 0.2)
            exbuf[b, 1, sl] = jnp.where(okv, jnp.exp(e1), 0.0)
        for g in range(8):
            e0v = exbuf[b, 0, pl.ds(g * 16, 16)]
            e1v = exbuf[b, 1, pl.ds(g * 16, 16)]
            for t in range(16):
                k = g * 16 + t
                x0 = jnp.full((16,), e0v[t], jnp.float32)
                x1 = jnp.full((16,), e1v[t], jnp.float32)
                for j in range(4):
                    sl2 = pl.ds(j * 16, 16)
                    msgbuf[b, k, sl2] = msgbuf[b, k, sl2] * (x0 if j < 2 else x1)
        pltpu.async_copy(msgbuf.at[b], acc.at[lidxbuf.at[b]], sem_s[b],
                         add=True)
        pltpu.async_copy(exbuf.at[b, 0], accd.at[exidx.at[b, 0]], sem_s[b],
                         add=True)
        pltpu.async_copy(exbuf.at[b, 1], accd.at[exidx.at[b, 1]], sem_s[b],
                         add=True)

    def drain_scatter(b):
        pltpu.make_async_copy(table.at[pl.ds(0, _CH)], msgbuf.at[b],
                              sem_s[b]).wait()
        pltpu.make_async_copy(as0.at[pl.ds(0, _CH)], exbuf.at[b, 0],
                              sem_s[b]).wait()
        pltpu.make_async_copy(as0.at[pl.ds(0, _CH)], exbuf.at[b, 1],
                              sem_s[b]).wait()

    issue_idx(0, 0)
    issue_idx(1, 1)
    drain_idx(0)
    issue_gathers(0)

    def body2(m, _):
        for b in (0, 1):
            i = 2 * m + b
            drain_gathers(b)

            @pl.when(i + 1 < _CPT)
            def _():
                drain_idx(1 - b)

                @pl.when(i >= 1)
                def _():
                    drain_scatter(1 - b)

                issue_gathers(1 - b)

            compute(i, b)

            @pl.when(i + 2 < _CPT)
            def _():
                issue_idx(i + 2, b)
        return ()

    lax.fori_loop(0, _CPT // 2, body2, (), unroll=False)
    drain_scatter(0)
    drain_scatter(1)

    plsc.subcore_barrier()
    rbase = s * _RPT
    pltpu.sync_copy(acc.at[pl.ds(rbase, _RPT)],
                    num_out.at[pl.ds(c * _ACCR + rbase, _RPT)])
    for h in range(2):
        pltpu.sync_copy(
            accd.at[pl.ds(h * _ACCR + rbase, _RPT)],
            den_out.at[pl.ds((c * 2 + h) * _ACCR + rbase, _RPT)])


@functools.partial(
    pl.kernel,
    out_type=[
        jax.ShapeDtypeStruct((2 * _ACCR, _HID), jnp.float32),
        jax.ShapeDtypeStruct((4 * _ACCR,), jnp.float32),
    ],
    mesh=plsc.VectorSubcoreMesh(core_axis_name="c", subcore_axis_name="s"),
    scratch_types=[
        pltpu.VMEM_SHARED((_ACCR, _HID), jnp.float32),
        pltpu.VMEM_SHARED((2 * _ACCR,), jnp.float32),
        pltpu.VMEM((2, _CH), jnp.int32),
        pltpu.VMEM((2, _CH), jnp.int32),
        pltpu.VMEM((2, _CH), jnp.int32),
        pltpu.VMEM((2, _CH), jnp.float32),
        pltpu.VMEM((2, _CH), jnp.float32),
        pltpu.VMEM((2, _CH), jnp.float32),
        pltpu.VMEM((2, _CH), jnp.float32),
        pltpu.VMEM((2, 2, _CH), jnp.float32),
        pltpu.VMEM((2, 2, _CH), jnp.int32),
        pltpu.VMEM((2, _CH, _HID), jnp.float32),
        pltpu.SemaphoreType.DMA,
        pltpu.SemaphoreType.DMA,
        pltpu.SemaphoreType.DMA,
        pltpu.SemaphoreType.DMA,
        pltpu.SemaphoreType.DMA,
        pltpu.SemaphoreType.DMA,
        pltpu.SemaphoreType.DMA,
        pltpu.SemaphoreType.DMA,
    ],
    compiler_params=pltpu.CompilerParams(use_tc_tiling_on_sc=False),
)
def _edge_pass(*refs):
    _edge_body(*refs)


def _att_mat(att):
    a = jnp.zeros((_HID, _H), jnp.float32)
    return a.at[:_C, 0].set(att[0]).at[_C:, 1].set(att[1])


def _v_ext(p_src, p_dst):
    vs = p_src['W_src'] @ _att_mat(p_src['att_src'])
    vd = p_dst['W_dst'] @ _att_mat(p_dst['att_dst'])
    return jnp.concatenate([vs, vd, jnp.zeros((_HID, 4), jnp.float32)], axis=1)


def _pad_edges(ei):
    pad = jnp.zeros((_EPAD - _E,), jnp.int32)
    return (jnp.concatenate([ei[0], pad]), jnp.concatenate([ei[1], pad]))


def _unbank(num_raw, den_raw):
    num = jnp.concatenate([num_raw[:_HALF], num_raw[_ACCR:_ACCR + _HALF]])
    dr = den_raw.reshape(2, 2, _ACCR)
    den = jnp.concatenate([dr[0, :, :_HALF], dr[1, :, :_HALF]], axis=1).T
    return num, den


def kernel(x_user, x_tweet, params, edge_index_ut, edge_index_tu):
    p = params
    su, du = _pad_edges(edge_index_ut)
    st, dt = _pad_edges(edge_index_tu)
    znum = jnp.zeros((_RPT, _HID), jnp.float32)
    zden = jnp.zeros((2 * _RPT,), jnp.float32)
    c0u, c0t, c1u = p['conv0_ut'], p['conv0_tu'], p['conv1_ut']

    tab_u, a_u = _prep(x_user, p['W_emb_user'] @ c0u['W_src'],
                       p['b_emb_user'] @ c0u['W_src'],
                       p['W_emb_user'] @ _v_ext(c0u, c0t),
                       p['b_emb_user'] @ _v_ext(c0u, c0t))
    tab_t, a_t = _prep(x_tweet, p['W_emb_tweet'] @ c0t['W_src'],
                       p['b_emb_tweet'] @ c0t['W_src'],
                       p['W_emb_tweet'] @ _v_ext(c0t, c0u),
                       p['b_emb_tweet'] @ _v_ext(c0t, c0u))
    n_ut0, d_ut0 = _unbank(*_edge_pass(tab_u, a_u[:, 0], a_u[:, 1],
                                       a_t[:, 2], a_t[:, 3], su, du,
                                       znum, zden))
    n_tu0, d_tu0 = _unbank(*_edge_pass(tab_t, a_t[:, 0], a_t[:, 1],
                                       a_u[:, 2], a_u[:, 3], st, dt,
                                       znum, zden))

    tab1, a1s = _epi_prep(n_tu0, d_tu0, c0t['bias'], c1u['W_src'],
                          _v_ext(c1u, c1u))
    _, a1d = _epi_prep(n_ut0, d_ut0, c0u['bias'], c1u['W_src'],
                       _v_ext(c1u, c1u))
    n_ut1, d_ut1 = _unbank(*_edge_pass(tab1, a1s[:, 0], a1s[:, 1],
                                       a1d[:, 2], a1d[:, 3], su, du,
                                       znum, zden))

    return _epi_mm(n_ut1, d_ut1, c1u['bias'], p['W_out'], p['b_out'])
